# baseline (speedup 1.0000x reference)
import functools

import jax
import jax.numpy as jnp
from jax import lax
from jax.experimental import pallas as pl
from jax.experimental.pallas import tpu as pltpu
from jax.experimental.pallas import tpu_sc as plsc

DIM = 64
SEQ = 200
NUM_CLASSES = 128
NC = 2
NS = 16
NW = NC * NS

_C0, _C1 = 104, 96
NBUF = 4

_DCOL = 16384
_DROW = _DCOL // 2


def _detile_body(x_ref, eye_ref, o_ref):
  x = x_ref[:].astype(jnp.bfloat16)
  eye = eye_ref[:]
  o_ref[:, 0:DIM] = lax.dot_general(
      x[:, 0:_DROW], eye, (((0,), (0,)), ((), ())),
      preferred_element_type=jnp.float32)
  o_ref[:, DIM:2 * DIM] = lax.dot_general(
      x[:, _DROW:_DCOL], eye, (((0,), (0,)), ((), ())),
      preferred_element_type=jnp.float32)


def _detile(table):
  vocab = table.shape[0]
  grid = (vocab + _DCOL - 1) // _DCOL
  return pl.pallas_call(
      _detile_body,
      grid=(grid,),
      in_specs=[
          pl.BlockSpec((DIM, _DCOL), lambda i: (0, i)),
          pl.BlockSpec((DIM, DIM), lambda i: (0, 0)),
      ],
      out_specs=pl.BlockSpec((_DROW, 2 * DIM), lambda i: (i, 0)),
      out_shape=jax.ShapeDtypeStruct((grid * _DROW, 2 * DIM), jnp.float32),
  )(table.T, jnp.eye(DIM, dtype=jnp.bfloat16))


def _pool_body(ids_hbm, table_hbm, out_hbm, idx_v, rows_v, out_v, *sems):
  batch = out_hbm.shape[0]
  bpw = batch // NW
  wid = lax.axis_index("s") * NC + lax.axis_index("c")
  base = wid * bpw

  pltpu.sync_copy(ids_hbm.at[pl.ds(base, bpw)], idx_v)

  def start_row(i, b):
    pltpu.make_async_copy(
        table_hbm.at[idx_v.at[i, pl.ds(0, _C0)]],
        rows_v.at[b, pl.ds(0, _C0)], sems[b]).start()
    pltpu.make_async_copy(
        table_hbm.at[idx_v.at[i, pl.ds(_C0, _C1)]],
        rows_v.at[b, pl.ds(_C0, _C1)], sems[b]).start()

  def wait_row(b):
    pltpu.make_async_copy(table_hbm.at[pl.ds(0, SEQ)],
                          rows_v.at[b], sems[b]).wait()

  def accum_row(i, b):
    def jbody(jj, carry):
      a = list(carry)
      j = jj * 4
      for u in range(4):
        for k in range(4):
          a[k] = a[k] + rows_v[b, j + u, pl.ds(16 * k, 16)]
      return tuple(a)
    acc = lax.fori_loop(
        0, SEQ // 4, jbody,
        tuple(jnp.zeros((16,), jnp.float32) for _ in range(4)))
    for k in range(4):
      out_v[i, pl.ds(16 * k, 16)] = acc[k]

  for b in range(NBUF):
    start_row(b, b)

  def gbody(t, _):
    for b in range(NBUF):
      i = t * NBUF + b
      wait_row(b)
      accum_row(i, b)
      start_row(i + NBUF, b)
    return 0

  lax.fori_loop(0, (bpw - NBUF) // NBUF, gbody, 0)
  for b in range(NBUF):
    wait_row(b)
    accum_row(bpw - NBUF + b, b)

  pltpu.sync_copy(out_v, out_hbm.at[pl.ds(base, bpw)])


def _pool(ids_mapped, table_flat):
  batch = ids_mapped.shape[0]
  bpw = batch // NW
  mesh = plsc.VectorSubcoreMesh(core_axis_name="c", subcore_axis_name="s")
  k = functools.partial(
      pl.kernel,
      out_type=jax.ShapeDtypeStruct((batch, DIM), jnp.float32),
      mesh=mesh,
      scratch_types=[
          pltpu.VMEM((bpw, SEQ), jnp.int32),
          pltpu.VMEM((NBUF, SEQ, DIM), jnp.float32),
          pltpu.VMEM((bpw, DIM), jnp.float32),
      ] + [pltpu.SemaphoreType.DMA] * NBUF,
      compiler_params=pltpu.CompilerParams(use_tc_tiling_on_sc=False),
  )(_pool_body)
  return k(ids_mapped, table_flat)


def _mlp_body(x_ref, w1_ref, b1_ref, w2_ref, b2_ref, out_ref):
  x = x_ref[:]
  h = jnp.tanh(
      lax.dot_general(x, w1_ref[:], (((1,), (1,)), ((), ())),
                      preferred_element_type=jnp.float32) + b1_ref[:])
  out_ref[:] = lax.dot_general(
      h, w2_ref[:], (((1,), (1,)), ((), ())),
      preferred_element_type=jnp.float32) + b2_ref[:]


def _mlp(pooled, W1, b1, W2, b2):
  batch = pooled.shape[0]
  blk = 1024
  return pl.pallas_call(
      _mlp_body,
      grid=(batch // blk,),
      in_specs=[
          pl.BlockSpec((blk, DIM), lambda i: (i, 0)),
          pl.BlockSpec((DIM, DIM), lambda i: (0, 0)),
          pl.BlockSpec((1, DIM), lambda i: (0, 0)),
          pl.BlockSpec((NUM_CLASSES, DIM), lambda i: (0, 0)),
          pl.BlockSpec((1, NUM_CLASSES), lambda i: (0, 0)),
      ],
      out_specs=pl.BlockSpec((blk, NUM_CLASSES), lambda i: (i, 0)),
      out_shape=jax.ShapeDtypeStruct((batch, NUM_CLASSES), jnp.float32),
  )(pooled, W1, b1.reshape(1, DIM), W2, b2.reshape(1, NUM_CLASSES))


def kernel(word_ids, table, W1, b1, W2, b2):
  ids = word_ids.astype(jnp.int32)
  ids_mapped = ((ids // _DCOL) * _DCOL + 2 * (ids % _DROW)
                + ((ids // _DROW) & 1))
  table2 = _detile(table)
  table_flat = table2.reshape(table2.shape[0] * 2, DIM)
  pooled = _pool(ids_mapped, table_flat)
  return _mlp(pooled, W1, b1, W2, b2)
